# Initial kernel scaffold; baseline (speedup 1.0000x reference)
import jax
import jax.numpy as jnp
from jax.experimental import pallas as pl


def kernel(feat, anchor, target_anchor):
    raise NotImplementedError("write your pallas kernel here")


# speedup vs baseline: 22.2555x; 22.2555x over previous
import jax
import jax.numpy as jnp
from jax.experimental import pallas as pl

KNN = 10
QB = 256
N_ANCHOR = 16384
D = 128


def _knn_block_kernel(feat_ref, anchor_t_ref, target_ref, out_ref):
    feat = feat_ref[...]
    at = anchor_t_ref[...]
    b2 = jnp.sum(at * at, axis=0, keepdims=True)
    s = b2 - 2.0 * jnp.dot(feat, at, preferred_element_type=jnp.float32)
    col = jax.lax.broadcasted_iota(jnp.int32, s.shape, 1)
    sel = jnp.zeros_like(s)
    for _ in range(KNN):
        v = jnp.min(s, axis=1, keepdims=True)
        eq = s == v
        j = jnp.min(jnp.where(eq, col, N_ANCHOR), axis=1, keepdims=True)
        onehot = col == j
        sel = sel + onehot.astype(jnp.float32)
        s = jnp.where(onehot, jnp.inf, s)
    out_ref[...] = jnp.dot(sel, target_ref[...],
                           preferred_element_type=jnp.float32) * (1.0 / KNN)


def kernel(feat, anchor, target_anchor):
    q = feat.shape[0]
    anchor_t = anchor.T
    grid = (q // QB,)
    return pl.pallas_call(
        _knn_block_kernel,
        grid=grid,
        in_specs=[
            pl.BlockSpec((QB, D), lambda i: (i, 0)),
            pl.BlockSpec((D, N_ANCHOR), lambda i: (0, 0)),
            pl.BlockSpec((N_ANCHOR, D), lambda i: (0, 0)),
        ],
        out_specs=pl.BlockSpec((QB, D), lambda i: (i, 0)),
        out_shape=jax.ShapeDtypeStruct((q, D), jnp.float32),
    )(feat, anchor_t, target_anchor)


# speedup vs baseline: 28.9209x; 1.2995x over previous
import functools

import jax
import jax.numpy as jnp
from jax import lax
from jax.experimental import pallas as pl
from jax.experimental.pallas import tpu as pltpu
from jax.experimental.pallas import tpu_sc as plsc

KNN = 10
QB = 256
N_ANCHOR = 16384
D = 128
NQ = 4096

NC, NS = 2, 16
NW = NC * NS
QPW = NQ // NW
CH = 8
NCH = QPW // CH


def _topk_kernel(feat_ref, anchor_t_ref, idx_ref):
    feat = feat_ref[...]
    at = anchor_t_ref[...]
    b2 = jnp.sum(at * at, axis=0, keepdims=True)
    s = b2 - 2.0 * jnp.dot(feat, at, preferred_element_type=jnp.float32)
    col = jax.lax.broadcasted_iota(jnp.int32, s.shape, 1)
    js = []
    for _ in range(KNN):
        v = jnp.min(s, axis=1, keepdims=True)
        eq = s == v
        j = jnp.min(jnp.where(eq, col, N_ANCHOR), axis=1, keepdims=True)
        js.append(j)
        s = jnp.where(col == j, jnp.inf, s)
    js.append(jnp.zeros((QB, 16 - KNN), jnp.int32))
    idx_ref[...] = jnp.concatenate(js, axis=1)


def _topk_indices(feat, anchor):
    anchor_t = anchor.T
    return pl.pallas_call(
        _topk_kernel,
        grid=(NQ // QB,),
        in_specs=[
            pl.BlockSpec((QB, D), lambda i: (i, 0)),
            pl.BlockSpec((D, N_ANCHOR), lambda i: (0, 0)),
        ],
        out_specs=pl.BlockSpec((QB, 16), lambda i: (i, 0)),
        out_shape=jax.ShapeDtypeStruct((NQ, 16), jnp.int32),
    )(feat, anchor_t)


@functools.partial(
    pl.kernel,
    out_type=jax.ShapeDtypeStruct((NQ, D), jnp.float32),
    mesh=plsc.VectorSubcoreMesh(core_axis_name="c", subcore_axis_name="s"),
    scratch_types=[
        pltpu.VMEM((2, CH * KNN), jnp.int32),
        pltpu.VMEM((2, CH * KNN, D), jnp.float32),
        pltpu.VMEM((QPW, D), jnp.float32),
        pltpu.SemaphoreType.DMA,
        pltpu.SemaphoreType.DMA,
    ],
)
def _gather_mean(idx_hbm, tgt_hbm, out_hbm, idx_v, rows_v, out_v, sem0, sem1):
    wid = lax.axis_index("s") * NC + lax.axis_index("c")
    sems = (sem0, sem1)
    copies = {}

    def fire(c):
        b = c % 2
        flat = (wid * QPW + c * CH) * KNN
        pltpu.sync_copy(idx_hbm.at[pl.ds(flat, CH * KNN)], idx_v.at[b])
        copies[c] = pltpu.async_copy(tgt_hbm.at[idx_v.at[b]], rows_v.at[b],
                                     sems[b])

    fire(0)
    for c in range(NCH):
        if c + 1 < NCH:
            fire(c + 1)
        copies[c].wait()
        b = c % 2

        def body(q, _, b=b, c=c):
            r0 = q * KNN
            for d in range(D // 16):
                sl = pl.ds(d * 16, 16)
                acc = rows_v[b, r0, sl]
                for r in range(1, KNN):
                    acc = acc + rows_v[b, r0 + r, sl]
                out_v[c * CH + q, sl] = acc * (1.0 / KNN)
            return ()

        lax.fori_loop(0, CH, body, ())

    pltpu.sync_copy(out_v, out_hbm.at[pl.ds(wid * QPW, QPW), :])


def kernel(feat, anchor, target_anchor):
    idx = _topk_indices(feat, anchor)
    idx_flat = idx[:, :KNN].reshape(-1)
    return _gather_mean(idx_flat, target_anchor)
